# Initial kernel scaffold; baseline (speedup 1.0000x reference)
import jax
import jax.numpy as jnp
from jax.experimental import pallas as pl


def kernel(semantic_features_list, adjacency_matrices_list, W_ih, W_hh, b_ih, b_hh, gx_w1, gx_b1, gx_w2, gx_b2, gz_w1, gz_b1, gz_w2, gz_b2, gv_w1, gv_b1, gv_w2, gv_b2, gn_w1, gn_b1, gn_w2, gn_b2, op_w, op_b):
    raise NotImplementedError("write your pallas kernel here")


# speedup vs baseline: 4.4747x; 4.4747x over previous
import functools

import jax
import jax.numpy as jnp
from jax.experimental import pallas as pl
from jax.experimental.pallas import tpu as pltpu

N = 4096
D = 256
H = 512
M = 256
O = 256

T_BLK = 512
R_BLK = 512
A_BLK = 256


def _dot_t(x, w):
    return jax.lax.dot_general(x, w, (((1,), (1,)), ((), ())),
                               preferred_element_type=jnp.float32)


def _lstm_kernel(x_ref, wih_ref, whh_ref, b_ref, s_ref, h_ref, c_ref, pre_ref):
    @pl.when(pl.program_id(0) == 0)
    def _():
        h_ref[...] = jnp.zeros_like(h_ref)
        c_ref[...] = jnp.zeros_like(c_ref)

    pre_ref[...] = _dot_t(x_ref[...], wih_ref[...]) + b_ref[...]

    def body(t, carry):
        h, c = carry
        gates = pre_ref[pl.ds(t, 1), :] + _dot_t(h, whh_ref[...])
        i_g = jax.nn.sigmoid(gates[:, 0 * H:1 * H])
        f_g = jax.nn.sigmoid(gates[:, 1 * H:2 * H])
        g_g = jnp.tanh(gates[:, 2 * H:3 * H])
        o_g = jax.nn.sigmoid(gates[:, 3 * H:4 * H])
        c = f_g * c + i_g * g_g
        h = o_g * jnp.tanh(c)
        s_ref[pl.ds(t, 1), :] = h
        return (h, c)

    h, c = jax.lax.fori_loop(0, T_BLK, body, (h_ref[...], c_ref[...]))
    h_ref[...] = h
    c_ref[...] = c


def _lstm_call(x, W_ih, W_hh, b):
    grid = (N // T_BLK,)
    return pl.pallas_call(
        _lstm_kernel,
        grid=grid,
        in_specs=[
            pl.BlockSpec((T_BLK, D), lambda t: (t, 0)),
            pl.BlockSpec((4 * H, D), lambda t: (0, 0)),
            pl.BlockSpec((4 * H, H), lambda t: (0, 0)),
            pl.BlockSpec((1, 4 * H), lambda t: (0, 0)),
        ],
        out_specs=pl.BlockSpec((T_BLK, H), lambda t: (t, 0)),
        out_shape=jax.ShapeDtypeStruct((N, H), jnp.float32),
        scratch_shapes=[
            pltpu.VMEM((1, H), jnp.float32),
            pltpu.VMEM((1, H), jnp.float32),
            pltpu.VMEM((T_BLK, 4 * H), jnp.float32),
        ],
        compiler_params=pltpu.CompilerParams(
            dimension_semantics=("arbitrary",)),
    )(x, W_ih, W_hh, b)


def _gxv_kernel(s_ref, xw1_ref, xb1_ref, xw2_ref, xb2_ref,
                vw1_ref, vb1_ref, vw2_ref, vb2_ref, gx_ref, gv_ref):
    s = s_ref[...]
    hx = jax.nn.relu(_dot_t(s, xw1_ref[...]) + xb1_ref[...])
    gx_ref[...] = _dot_t(hx, xw2_ref[...]) + xb2_ref[...]
    hv = jax.nn.relu(_dot_t(s, vw1_ref[...]) + vb1_ref[...])
    gv_ref[...] = _dot_t(hv, vw2_ref[...]) + vb2_ref[...]


def _gxv_call(S, gx_w1, gx_b1, gx_w2, gx_b2, gv_w1, gv_b1, gv_w2, gv_b2):
    grid = (N // R_BLK,)
    full = lambda t: (0, 0)
    row = lambda t: (t, 0)
    return pl.pallas_call(
        _gxv_kernel,
        grid=grid,
        in_specs=[
            pl.BlockSpec((R_BLK, H), row),
            pl.BlockSpec((M, H), full), pl.BlockSpec((1, M), full),
            pl.BlockSpec((M, M), full), pl.BlockSpec((1, M), full),
            pl.BlockSpec((M, H), full), pl.BlockSpec((1, M), full),
            pl.BlockSpec((M, M), full), pl.BlockSpec((1, M), full),
        ],
        out_specs=[pl.BlockSpec((R_BLK, M), row),
                   pl.BlockSpec((R_BLK, M), row)],
        out_shape=[jax.ShapeDtypeStruct((N, M), jnp.float32),
                   jax.ShapeDtypeStruct((N, M), jnp.float32)],
        compiler_params=pltpu.CompilerParams(
            dimension_semantics=("parallel",)),
    )(S, gx_w1, gx_b1, gx_w2, gx_b2, gv_w1, gv_b1, gv_w2, gv_b2)


def _attn_kernel(gx_ref, gv_ref, s_ref, adj_ref,
                 zw1_ref, zb1_ref, zw2_ref, zb2_ref,
                 nw1_ref, nb1_ref, nw2_ref, nb2_ref,
                 ops_ref, opa_ref, opb_ref, out_ref):
    i = pl.program_id(0)
    row0 = i * A_BLK

    scores = _dot_t(gx_ref[...], gv_ref[...])
    col = jax.lax.broadcasted_iota(jnp.int32, (A_BLK, N), 1)
    rowid = row0 + jax.lax.broadcasted_iota(jnp.int32, (A_BLK, N), 0)
    mask = (adj_ref[...] > 0) & (col != rowid)

    neg = jnp.float32(-1e30)
    masked = jnp.where(mask, scores, neg)
    m = jnp.max(masked, axis=1, keepdims=True)
    p = jnp.where(mask, jnp.exp(scores - m), 0.0)
    denom = jnp.sum(p, axis=1, keepdims=True)
    has_nb = denom > 0.0
    attn = p / jnp.where(has_nb, denom, 1.0)

    summed = jnp.dot(attn, s_ref[...],
                     preferred_element_type=jnp.float32)

    hz = jax.nn.relu(_dot_t(summed, zw1_ref[...]) + zb1_ref[...])
    nb_enc = _dot_t(hz, zw2_ref[...]) + zb2_ref[...]
    hn = jax.nn.relu(_dot_t(nb_enc, nw1_ref[...]) + nb1_ref[...])
    agg = _dot_t(hn, nw2_ref[...]) + nb2_ref[...]
    agg = jnp.where(has_nb, agg, 0.0)

    s_blk = s_ref[pl.ds(row0, A_BLK), :]
    out_ref[...] = (_dot_t(s_blk, ops_ref[...]) + _dot_t(agg, opa_ref[...])
                    + opb_ref[...])


def _attn_call(gx, gv, S, adj, gz_w1, gz_b1, gz_w2, gz_b2,
               gn_w1, gn_b1, gn_w2, gn_b2, op_ws, op_wa, op_b):
    grid = (N // A_BLK,)
    full = lambda t: (0, 0)
    row = lambda t: (t, 0)
    return pl.pallas_call(
        _attn_kernel,
        grid=grid,
        in_specs=[
            pl.BlockSpec((A_BLK, M), row),
            pl.BlockSpec((N, M), full),
            pl.BlockSpec((N, H), full),
            pl.BlockSpec((A_BLK, N), row),
            pl.BlockSpec((M, H), full), pl.BlockSpec((1, M), full),
            pl.BlockSpec((M, M), full), pl.BlockSpec((1, M), full),
            pl.BlockSpec((M, M), full), pl.BlockSpec((1, M), full),
            pl.BlockSpec((O, M), full), pl.BlockSpec((1, O), full),
            pl.BlockSpec((O, H), full), pl.BlockSpec((O, O), full),
            pl.BlockSpec((1, O), full),
        ],
        out_specs=pl.BlockSpec((A_BLK, O), row),
        out_shape=jax.ShapeDtypeStruct((N, O), jnp.float32),
        compiler_params=pltpu.CompilerParams(
            dimension_semantics=("arbitrary",)),
    )(gx, gv, S, adj, gz_w1, gz_b1, gz_w2, gz_b2,
      gn_w1, gn_b1, gn_w2, gn_b2, op_ws, op_wa, op_b)


@jax.jit
def kernel(semantic_features_list, adjacency_matrices_list, W_ih, W_hh,
           b_ih, b_hh, gx_w1, gx_b1, gx_w2, gx_b2, gz_w1, gz_b1, gz_w2,
           gz_b2, gv_w1, gv_b1, gv_w2, gv_b2, gn_w1, gn_b1, gn_w2, gn_b2,
           op_w, op_b):
    b = (b_ih + b_hh).reshape(1, 4 * H)
    op_ws = op_w[:, :H]
    op_wa = op_w[:, H:]
    outs = []
    for g in range(semantic_features_list.shape[0]):
        x = semantic_features_list[g]
        adj = adjacency_matrices_list[g]
        S = _lstm_call(x, W_ih, W_hh, b)
        gx, gv = _gxv_call(S, gx_w1, gx_b1.reshape(1, M), gx_w2,
                           gx_b2.reshape(1, M), gv_w1, gv_b1.reshape(1, M),
                           gv_w2, gv_b2.reshape(1, M))
        out = _attn_call(gx, gv, S, adj,
                         gz_w1, gz_b1.reshape(1, M), gz_w2,
                         gz_b2.reshape(1, M), gn_w1, gn_b1.reshape(1, M),
                         gn_w2, gn_b2.reshape(1, M), op_ws, op_wa,
                         op_b.reshape(1, O))
        outs.append(out)
    return jnp.stack(outs, axis=0)
